# baseline (speedup 1.0000x reference)
import math

import jax
import jax.numpy as jnp
from jax.experimental import pallas as pl
from jax.experimental.pallas import tpu as pltpu

_K = 256
_M = 100000
_C = 100
_D = 64
_B = 1024
_ALPHA = 0.1
_TEMP = max(1.0, float(math.log(0.2 * _K)))
_NEG = -1e9

_MB = 2048
_NSTEPS = 49
_M_PAD = _MB * _NSTEPS
_RB = 16
_NRB = _B // _RB

_KEY_LO = -1073741826
_KEY_HI = 1069547520


def _key_to_f32(k):
    return jax.lax.bitcast_convert_type(k ^ ((k >> 31) & jnp.int32(0x7FFFFFFF)),
                                        jnp.float32)


def _sims_kernel(x_ref, k_ref, out_ref):
    j = pl.program_id(0)
    x = x_ref[...]
    nx = x / jnp.sqrt(jnp.sum(x * x, axis=1, keepdims=True) + 1e-12)
    k = k_ref[...]
    nk = k / jnp.sqrt(jnp.sum(k * k, axis=1, keepdims=True) + 1e-12)
    s = jax.lax.dot_general(nx, nk, (((1,), (1,)), ((), ())),
                            preferred_element_type=jnp.float32)
    col = j * _MB + jax.lax.broadcasted_iota(jnp.int32, s.shape, 1)
    out_ref[...] = jnp.where(col < _M, s, jnp.float32(-2.0))


def _count_ge(s, mid):
    midf = _key_to_f32(mid)
    n = s.shape[1]
    cw = 1792 if n % 1792 == 0 else n
    acc = jnp.zeros((s.shape[0], cw), jnp.int32)
    for c in range(n // cw):
        acc = jnp.where(s[:, c * cw:(c + 1) * cw] >= midf, acc + 1, acc)
    return jnp.sum(acc, axis=1, keepdims=True)


def _count_ge_ref(s_ref, mid):
    midf = _key_to_f32(mid)
    cw = 3584
    ones = jnp.ones((cw, 128), jnp.float32)
    acc = jnp.zeros((s_ref.shape[0], 128), jnp.float32)
    for c in range(s_ref.shape[1] // cw):
        m = jnp.where(s_ref[:, c * cw:(c + 1) * cw] >= midf, 1.0, 0.0)
        acc = acc + jax.lax.dot_general(m, ones, (((1,), (0,)), ((), ())),
                                        preferred_element_type=jnp.float32)
    return acc[:, 0:1].astype(jnp.int32)


def _thresh_kernel(s_ref, t_ref):
    g = _M_PAD // 8
    p = s_ref[:, 0:g]
    for i in range(1, 8):
        p = jnp.maximum(p, s_ref[:, i * g:(i + 1) * g])
    rowmax = jnp.max(p, axis=1, keepdims=True)

    def sub_body(_, carry):
        lo, hi = carry
        mid = lo + ((hi - lo + 1) >> 1)
        ge = _count_ge(p, mid) >= _K
        return jnp.where(ge, mid, lo), jnp.where(ge, hi, mid - 1)

    tau, _ = jax.lax.fori_loop(
        0, 25, sub_body,
        (jnp.full((_RB, 1), _KEY_LO, jnp.int32),
         jnp.full((_RB, 1), _KEY_HI, jnp.int32)))

    hi0 = jax.lax.bitcast_convert_type(rowmax, jnp.int32)
    hi0 = hi0 ^ ((hi0 >> 31) & jnp.int32(0x7FFFFFFF))

    def main_cond(carry):
        lo, hi = carry
        return jnp.any(lo < hi)

    def main_body(carry):
        lo, hi = carry
        mid = lo + ((hi - lo + 1) >> 1)
        ge = _count_ge_ref(s_ref, mid) >= _K
        return jnp.where(ge, mid, lo), jnp.where(ge, hi, mid - 1)

    lo, _ = jax.lax.while_loop(main_cond, main_body, (tau, hi0))
    t_ref[...] = jnp.broadcast_to(_key_to_f32(lo), (_RB, 128))


def _final_kernel(s_ref, lab_ref, t_ref, y_ref,
                  probs_ref, closest_ref, teacher_ref,
                  denom_ref, pos_ref, neg_ref, rmax_ref, rlab_ref):
    j = pl.program_id(0)

    @pl.when(j == 0)
    def _init():
        probs_ref[...] = jnp.zeros_like(probs_ref)
        denom_ref[...] = jnp.zeros_like(denom_ref)
        pos_ref[...] = jnp.full_like(pos_ref, _NEG)
        neg_ref[...] = jnp.full_like(neg_ref, _NEG)
        rmax_ref[...] = jnp.full_like(rmax_ref, -3.0)
        rlab_ref[...] = jnp.zeros_like(rlab_ref)

    s = s_ref[...]
    labels = lab_ref[0, :, :]
    t = t_ref[:, 0:1]
    y = y_ref[...]

    sel = s >= t
    w = jnp.where(sel, jnp.exp(s * _TEMP), 0.0)
    denom_ref[...] += jnp.sum(w, axis=1, keepdims=True)

    cls = jax.lax.broadcasted_iota(jnp.int32, (_C, _MB), 0)
    onehot_t = (cls == labels).astype(jnp.float32)
    probs_ref[...] += jax.lax.dot_general(
        w, onehot_t, (((1,), (1,)), ((), ())),
        preferred_element_type=jnp.float32)

    ymatch = labels == y
    posv = jnp.where(sel & ymatch, s, _NEG)
    negv = jnp.where(sel & jnp.logical_not(ymatch), s, _NEG)
    pos_ref[...] = jnp.maximum(pos_ref[...],
                               jnp.max(posv, axis=1, keepdims=True))
    neg_ref[...] = jnp.maximum(neg_ref[...],
                               jnp.max(negv, axis=1, keepdims=True))

    bmax = jnp.max(s, axis=1, keepdims=True)
    blab = jnp.min(jnp.where(s == bmax, jnp.broadcast_to(labels, s.shape),
                             jnp.int32(2**30)),
                   axis=1, keepdims=True)
    upd = bmax > rmax_ref[...]
    rlab_ref[...] = jnp.where(upd, blab, rlab_ref[...])
    rmax_ref[...] = jnp.maximum(rmax_ref[...], bmax)

    @pl.when(j == _NSTEPS - 1)
    def _fin():
        probs_ref[...] = probs_ref[...] / denom_ref[...]
        closest_ref[...] = rlab_ref[...]
        posm = pos_ref[...]
        negm = neg_ref[...]
        posf = jnp.where(posm > _NEG * 0.5, posm, 0.0)
        negf = jnp.where(negm > _NEG * 0.5, negm, 0.0)
        teacher_ref[...] = jnp.maximum(negf - posf + _ALPHA, 0.0)


def _impl(x, y, mem_keys, mem_vals):
    keys_pad = jnp.pad(mem_keys, ((0, _M_PAD - _M), (0, 0)))
    vals_pad = jnp.pad(mem_vals, (0, _M_PAD - _M)).reshape(_NSTEPS, 1, _MB)
    y2 = y.reshape(_B, 1)

    sims = pl.pallas_call(
        _sims_kernel,
        grid=(_NSTEPS,),
        in_specs=[
            pl.BlockSpec((_B, _D), lambda j: (0, 0)),
            pl.BlockSpec((_MB, _D), lambda j: (j, 0)),
        ],
        out_specs=pl.BlockSpec((_B, _MB), lambda j: (0, j)),
        out_shape=jax.ShapeDtypeStruct((_B, _M_PAD), jnp.float32),
        compiler_params=pltpu.CompilerParams(
            dimension_semantics=("arbitrary",)),
    )(x, keys_pad)

    thresh = pl.pallas_call(
        _thresh_kernel,
        grid=(_NRB,),
        in_specs=[pl.BlockSpec((_RB, _M_PAD), lambda i: (i, 0))],
        out_specs=pl.BlockSpec((_RB, 128), lambda i: (i, 0)),
        out_shape=jax.ShapeDtypeStruct((_B, 128), jnp.float32),
        compiler_params=pltpu.CompilerParams(
            dimension_semantics=("arbitrary",)),
    )(sims)

    probs, closest, teacher = pl.pallas_call(
        _final_kernel,
        grid=(_NSTEPS,),
        in_specs=[
            pl.BlockSpec((_B, _MB), lambda j: (0, j)),
            pl.BlockSpec((1, 1, _MB), lambda j: (j, 0, 0)),
            pl.BlockSpec((_B, 128), lambda j: (0, 0)),
            pl.BlockSpec((_B, 1), lambda j: (0, 0)),
        ],
        out_specs=[
            pl.BlockSpec((_B, _C), lambda j: (0, 0)),
            pl.BlockSpec((_B, 1), lambda j: (0, 0)),
            pl.BlockSpec((_B, 1), lambda j: (0, 0)),
        ],
        out_shape=[
            jax.ShapeDtypeStruct((_B, _C), jnp.float32),
            jax.ShapeDtypeStruct((_B, 1), jnp.int32),
            jax.ShapeDtypeStruct((_B, 1), jnp.float32),
        ],
        scratch_shapes=[
            pltpu.VMEM((_B, 1), jnp.float32),
            pltpu.VMEM((_B, 1), jnp.float32),
            pltpu.VMEM((_B, 1), jnp.float32),
            pltpu.VMEM((_B, 1), jnp.float32),
            pltpu.VMEM((_B, 1), jnp.int32),
        ],
        compiler_params=pltpu.CompilerParams(
            dimension_semantics=("arbitrary",)),
    )(sims, vals_pad, thresh, y2)

    return closest.reshape(_B), probs, teacher.reshape(_B)


kernel = jax.jit(_impl)


# speedup vs baseline: 3.1491x; 3.1491x over previous
import math

import jax
import jax.numpy as jnp
from jax.experimental import pallas as pl
from jax.experimental.pallas import tpu as pltpu

_K = 256
_M = 100000
_C = 100
_D = 64
_B = 1024
_ALPHA = 0.1
_TEMP = max(1.0, float(math.log(0.2 * _K)))
_NEG = -1e9

_MB = 2048
_NSTEPS = 49
_M_PAD = _MB * _NSTEPS
_RB = 16
_NRB = _B // _RB

_KEY_LO = -1073741826
_KEY_HI = 1069547520


def _key_to_f32(k):
    return jax.lax.bitcast_convert_type(k ^ ((k >> 31) & jnp.int32(0x7FFFFFFF)),
                                        jnp.float32)


def _sims_kernel(x_ref, k_ref, out_ref):
    j = pl.program_id(0)
    x = x_ref[...]
    nx = x / jnp.sqrt(jnp.sum(x * x, axis=1, keepdims=True) + 1e-12)
    k = k_ref[...]
    nk = k / jnp.sqrt(jnp.sum(k * k, axis=1, keepdims=True) + 1e-12)
    s = jax.lax.dot_general(nx, nk, (((1,), (1,)), ((), ())),
                            preferred_element_type=jnp.float32)
    col = j * _MB + jax.lax.broadcasted_iota(jnp.int32, s.shape, 1)
    out_ref[...] = jnp.where(col < _M, s, jnp.float32(-2.0))


def _count_ge(s, mid):
    midf = _key_to_f32(mid)
    n = s.shape[1]
    cw = 1792 if n % 1792 == 0 else n
    acc = jnp.zeros((s.shape[0], cw), jnp.int32)
    for c in range(n // cw):
        acc = jnp.where(s[:, c * cw:(c + 1) * cw] >= midf, acc + 1, acc)
    return jnp.sum(acc, axis=1, keepdims=True)


def _count_ge_ref(s_ref, mid):
    midf = _key_to_f32(mid)
    cw = 1792
    acc = jnp.zeros((s_ref.shape[0], cw), jnp.int32)
    for c in range(s_ref.shape[1] // cw):
        acc = jnp.where(s_ref[:, c * cw:(c + 1) * cw] >= midf, acc + 1, acc)
    return jnp.sum(acc, axis=1, keepdims=True)


def _thresh_kernel(s_ref, t_ref):
    g = _M_PAD // 8
    p = s_ref[:, 0:g]
    for i in range(1, 8):
        p = jnp.maximum(p, s_ref[:, i * g:(i + 1) * g])
    rowmax = jnp.max(p, axis=1, keepdims=True)

    def sub_body(_, carry):
        lo, hi = carry
        mid = lo + ((hi - lo + 1) >> 1)
        ge = _count_ge(p, mid) >= _K
        return jnp.where(ge, mid, lo), jnp.where(ge, hi, mid - 1)

    tau, _ = jax.lax.fori_loop(
        0, 25, sub_body,
        (jnp.full((_RB, 1), _KEY_LO, jnp.int32),
         jnp.full((_RB, 1), _KEY_HI, jnp.int32)))

    hi0 = jax.lax.bitcast_convert_type(rowmax, jnp.int32)
    hi0 = hi0 ^ ((hi0 >> 31) & jnp.int32(0x7FFFFFFF))

    def main_cond(carry):
        lo, hi = carry
        return jnp.any(lo < hi)

    def main_body(carry):
        lo, hi = carry
        mid = lo + ((hi - lo + 1) >> 1)
        ge = _count_ge_ref(s_ref, mid) >= _K
        return jnp.where(ge, mid, lo), jnp.where(ge, hi, mid - 1)

    lo, _ = jax.lax.while_loop(main_cond, main_body, (tau, hi0))
    t_ref[...] = jnp.broadcast_to(_key_to_f32(lo), (_RB, 128))


def _final_kernel(s_ref, lab_ref, t_ref, y_ref,
                  probs_ref, closest_ref, teacher_ref,
                  denom_ref, pos_ref, neg_ref, rmax_ref, rlab_ref):
    j = pl.program_id(0)

    @pl.when(j == 0)
    def _init():
        probs_ref[...] = jnp.zeros_like(probs_ref)
        denom_ref[...] = jnp.zeros_like(denom_ref)
        pos_ref[...] = jnp.full_like(pos_ref, _NEG)
        neg_ref[...] = jnp.full_like(neg_ref, _NEG)
        rmax_ref[...] = jnp.full_like(rmax_ref, -3.0)
        rlab_ref[...] = jnp.zeros_like(rlab_ref)

    s = s_ref[...]
    labels = lab_ref[0, :, :]
    t = t_ref[:, 0:1]
    y = y_ref[...]

    sel = s >= t
    w = jnp.where(sel, jnp.exp(s * _TEMP), 0.0)
    denom_ref[...] += jnp.sum(w, axis=1, keepdims=True)

    cls = jax.lax.broadcasted_iota(jnp.int32, (_C, _MB), 0)
    onehot_t = (cls == labels).astype(jnp.float32)
    probs_ref[...] += jax.lax.dot_general(
        w, onehot_t, (((1,), (1,)), ((), ())),
        preferred_element_type=jnp.float32)

    ymatch = labels == y
    posv = jnp.where(sel & ymatch, s, _NEG)
    negv = jnp.where(sel & jnp.logical_not(ymatch), s, _NEG)
    pos_ref[...] = jnp.maximum(pos_ref[...],
                               jnp.max(posv, axis=1, keepdims=True))
    neg_ref[...] = jnp.maximum(neg_ref[...],
                               jnp.max(negv, axis=1, keepdims=True))

    bmax = jnp.max(s, axis=1, keepdims=True)
    blab = jnp.min(jnp.where(s == bmax, jnp.broadcast_to(labels, s.shape),
                             jnp.int32(2**30)),
                   axis=1, keepdims=True)
    upd = bmax > rmax_ref[...]
    rlab_ref[...] = jnp.where(upd, blab, rlab_ref[...])
    rmax_ref[...] = jnp.maximum(rmax_ref[...], bmax)

    @pl.when(j == _NSTEPS - 1)
    def _fin():
        probs_ref[...] = probs_ref[...] / denom_ref[...]
        closest_ref[...] = rlab_ref[...]
        posm = pos_ref[...]
        negm = neg_ref[...]
        posf = jnp.where(posm > _NEG * 0.5, posm, 0.0)
        negf = jnp.where(negm > _NEG * 0.5, negm, 0.0)
        teacher_ref[...] = jnp.maximum(negf - posf + _ALPHA, 0.0)


def _impl(x, y, mem_keys, mem_vals):
    keys_pad = jnp.pad(mem_keys, ((0, _M_PAD - _M), (0, 0)))
    vals_pad = jnp.pad(mem_vals, (0, _M_PAD - _M)).reshape(_NSTEPS, 1, _MB)
    y2 = y.reshape(_B, 1)

    sims = pl.pallas_call(
        _sims_kernel,
        grid=(_NSTEPS,),
        in_specs=[
            pl.BlockSpec((_B, _D), lambda j: (0, 0)),
            pl.BlockSpec((_MB, _D), lambda j: (j, 0)),
        ],
        out_specs=pl.BlockSpec((_B, _MB), lambda j: (0, j)),
        out_shape=jax.ShapeDtypeStruct((_B, _M_PAD), jnp.float32),
        compiler_params=pltpu.CompilerParams(
            dimension_semantics=("arbitrary",)),
    )(x, keys_pad)

    thresh = pl.pallas_call(
        _thresh_kernel,
        grid=(_NRB,),
        in_specs=[pl.BlockSpec((_RB, _M_PAD), lambda i: (i, 0))],
        out_specs=pl.BlockSpec((_RB, 128), lambda i: (i, 0)),
        out_shape=jax.ShapeDtypeStruct((_B, 128), jnp.float32),
        compiler_params=pltpu.CompilerParams(
            dimension_semantics=("arbitrary",)),
    )(sims)

    probs, closest, teacher = pl.pallas_call(
        _final_kernel,
        grid=(_NSTEPS,),
        in_specs=[
            pl.BlockSpec((_B, _MB), lambda j: (0, j)),
            pl.BlockSpec((1, 1, _MB), lambda j: (j, 0, 0)),
            pl.BlockSpec((_B, 128), lambda j: (0, 0)),
            pl.BlockSpec((_B, 1), lambda j: (0, 0)),
        ],
        out_specs=[
            pl.BlockSpec((_B, _C), lambda j: (0, 0)),
            pl.BlockSpec((_B, 1), lambda j: (0, 0)),
            pl.BlockSpec((_B, 1), lambda j: (0, 0)),
        ],
        out_shape=[
            jax.ShapeDtypeStruct((_B, _C), jnp.float32),
            jax.ShapeDtypeStruct((_B, 1), jnp.int32),
            jax.ShapeDtypeStruct((_B, 1), jnp.float32),
        ],
        scratch_shapes=[
            pltpu.VMEM((_B, 1), jnp.float32),
            pltpu.VMEM((_B, 1), jnp.float32),
            pltpu.VMEM((_B, 1), jnp.float32),
            pltpu.VMEM((_B, 1), jnp.float32),
            pltpu.VMEM((_B, 1), jnp.int32),
        ],
        compiler_params=pltpu.CompilerParams(
            dimension_semantics=("arbitrary",)),
    )(sims, vals_pad, thresh, y2)

    return closest.reshape(_B), probs, teacher.reshape(_B)


kernel = jax.jit(_impl)
